# baseline (speedup 1.0000x reference)
import functools

import jax
import jax.numpy as jnp
import numpy as np
from jax.experimental import pallas as pl
from jax.experimental.pallas import tpu as pltpu
from jax.sharding import Mesh, PartitionSpec as P

try:
    from jax.experimental.shard_map import shard_map
except ImportError:
    from jax import shard_map


def _ldr_kernel(idx_ref, h0_ref, a2_ref, b2_ref, alog_ref, dtb_ref,
                qT_ref, kT_ref, v_ref, o_ref, *, T):
    h = h0_ref[0]
    x = a2_ref[0] + dtb_ref[:]
    sp = jnp.where(x <= 20.0, jnp.log1p(jnp.exp(jnp.minimum(x, 20.0))), x)
    gam = jnp.exp(-jnp.exp(alog_ref[:]) * sp)
    beta = jax.nn.sigmoid(b2_ref[0])
    qT = qT_ref[0]
    kT = kT_ref[0]

    def bf(z):
        return z.astype(jnp.bfloat16).astype(jnp.float32)

    for t in range(T):
        h = h * gam[:, t:t + 1][:, :, None]
        kcol = kT[:, :, t:t + 1]
        kv = jnp.sum(bf(kcol) * bf(h), axis=1)
        vres = (v_ref[0, t] - kv) * beta[:, t:t + 1]
        h = h + kcol * vres[:, None, :]
        o_ref[0, t] = jnp.sum(bf(qT[:, :, t:t + 1]) * bf(h), axis=1)


def _run_shard(idxs, src, a2, b2, alog, dtb, qT, kT, v2, *, T, HV, K, V):
    Bloc = idxs.shape[0]
    grid_spec = pltpu.PrefetchScalarGridSpec(
        num_scalar_prefetch=1,
        grid=(Bloc,),
        in_specs=[
            pl.BlockSpec((1, HV, K, V), lambda i, idx: (idx[i], 0, 0, 0)),
            pl.BlockSpec((1, HV, T), lambda i, idx: (i, 0, 0)),
            pl.BlockSpec((1, HV, T), lambda i, idx: (i, 0, 0)),
            pl.BlockSpec((HV, 1), lambda i, idx: (0, 0)),
            pl.BlockSpec((HV, 1), lambda i, idx: (0, 0)),
            pl.BlockSpec((1, HV, K, T), lambda i, idx: (i, 0, 0, 0)),
            pl.BlockSpec((1, HV, K, T), lambda i, idx: (i, 0, 0, 0)),
            pl.BlockSpec((1, T, HV, V), lambda i, idx: (i, 0, 0, 0)),
        ],
        out_specs=pl.BlockSpec((1, T, HV, V), lambda i, idx: (i, 0, 0, 0)),
    )
    body = functools.partial(_ldr_kernel, T=T)
    return pl.pallas_call(
        body,
        grid_spec=grid_spec,
        out_shape=jax.ShapeDtypeStruct((Bloc, T, HV, V), jnp.float32),
    )(idxs, src, a2, b2, alog, dtb, qT, kT, v2)


def kernel(A_log, a, dt_bias, q, k, v, b, initial_state_source, initial_state_indices):
    B, T, H, K = q.shape
    HV, V = v.shape[2], v.shape[3]
    rep = HV // H
    scale = K ** (-0.5)

    q_f = q.astype(jnp.float32)
    k_f = k.astype(jnp.float32)
    qT = jnp.repeat(q_f * scale, rep, axis=2).transpose(0, 2, 3, 1)
    kT = jnp.repeat(k_f, rep, axis=2).transpose(0, 2, 3, 1)
    v2 = v.astype(jnp.float32)
    a2 = a.astype(jnp.float32).reshape(B, T, HV).transpose(0, 2, 1)
    b2 = b.astype(jnp.float32).reshape(B, T, HV).transpose(0, 2, 1)
    alog = A_log.astype(jnp.float32).reshape(HV, 1)
    dtb = dt_bias.astype(jnp.float32).reshape(HV, 1)
    src = initial_state_source.astype(jnp.float32)

    run = functools.partial(_run_shard, T=T, HV=HV, K=K, V=V)
    devs = jax.devices()
    if len(devs) >= 2 and B % 2 == 0:
        mesh = Mesh(np.array(devs[:2]), ("d",))
        sharded = shard_map(
            run, mesh=mesh,
            in_specs=(P("d"), P(), P("d"), P("d"), P(), P(),
                      P("d"), P("d"), P("d")),
            out_specs=P("d"),
            check_rep=False,
        )
        o = sharded(initial_state_indices, src, a2, b2, alog, dtb, qT, kT, v2)
    else:
        o = run(initial_state_indices, src, a2, b2, alog, dtb, qT, kT, v2)
    return o.astype(v.dtype)


# speedup vs baseline: 2.5137x; 2.5137x over previous
import functools

import jax
import jax.numpy as jnp
from jax.experimental import pallas as pl
from jax.experimental.pallas import tpu as pltpu


def _ldr_kernel(idx_ref, h0_ref, a2_ref, b2_ref, alog_ref, dtb_ref,
                qT_ref, kT_ref, v_ref, o_ref, *, T):
    HV, K, V = h0_ref.shape[1], h0_ref.shape[2], h0_ref.shape[3]
    h = h0_ref[0]
    x = a2_ref[0] + dtb_ref[:]
    sp = jnp.where(x <= 20.0, jnp.log1p(jnp.exp(jnp.minimum(x, 20.0))), x)
    gam = jnp.exp(-jnp.exp(alog_ref[:]) * sp)
    beta = jax.nn.sigmoid(b2_ref[0])
    gamB = jnp.broadcast_to(gam[:, :, None], (HV, T, V))
    betaB = jnp.broadcast_to(beta[:, :, None], (HV, T, V))
    qT = qT_ref[0]
    kT = kT_ref[0]

    def bf(z):
        return z.astype(jnp.bfloat16).astype(jnp.float32)

    for t in range(T):
        h = h * gamB[:, t:t + 1, :]
        kcol = kT[:, :, t:t + 1]
        kv = jnp.sum(kcol * bf(h), axis=1)
        vres = (v_ref[0, t] - kv) * betaB[:, t, :]
        h = h + kcol * vres[:, None, :]
        o_ref[0, t] = jnp.sum(qT[:, :, t:t + 1] * bf(h), axis=1)


def kernel(A_log, a, dt_bias, q, k, v, b, initial_state_source, initial_state_indices):
    B, T, H, K = q.shape
    HV, V = v.shape[2], v.shape[3]
    rep = HV // H
    scale = K ** (-0.5)

    def bf(z):
        return z.astype(jnp.bfloat16).astype(jnp.float32)

    q_f = q.astype(jnp.float32)
    k_f = k.astype(jnp.float32)
    qT = jnp.repeat(bf(q_f * scale), rep, axis=2).transpose(0, 2, 3, 1)
    kT = jnp.repeat(bf(k_f), rep, axis=2).transpose(0, 2, 3, 1)
    v2 = v.astype(jnp.float32)
    a2 = a.astype(jnp.float32).reshape(B, T, HV).transpose(0, 2, 1)
    b2 = b.astype(jnp.float32).reshape(B, T, HV).transpose(0, 2, 1)
    alog = A_log.astype(jnp.float32).reshape(HV, 1)
    dtb = dt_bias.astype(jnp.float32).reshape(HV, 1)
    src = initial_state_source.astype(jnp.float32)

    grid_spec = pltpu.PrefetchScalarGridSpec(
        num_scalar_prefetch=1,
        grid=(B,),
        in_specs=[
            pl.BlockSpec((1, HV, K, V), lambda i, idx: (idx[i], 0, 0, 0)),
            pl.BlockSpec((1, HV, T), lambda i, idx: (i, 0, 0)),
            pl.BlockSpec((1, HV, T), lambda i, idx: (i, 0, 0)),
            pl.BlockSpec((HV, 1), lambda i, idx: (0, 0)),
            pl.BlockSpec((HV, 1), lambda i, idx: (0, 0)),
            pl.BlockSpec((1, HV, K, T), lambda i, idx: (i, 0, 0, 0)),
            pl.BlockSpec((1, HV, K, T), lambda i, idx: (i, 0, 0, 0)),
            pl.BlockSpec((1, T, HV, V), lambda i, idx: (i, 0, 0, 0)),
        ],
        out_specs=pl.BlockSpec((1, T, HV, V), lambda i, idx: (i, 0, 0, 0)),
    )
    body = functools.partial(_ldr_kernel, T=T)
    o = pl.pallas_call(
        body,
        grid_spec=grid_spec,
        out_shape=jax.ShapeDtypeStruct((B, T, HV, V), jnp.float32),
    )(initial_state_indices, src, a2, b2, alog, dtb, qT, kT, v2)
    return o.astype(v.dtype)
